# Initial kernel scaffold; baseline (speedup 1.0000x reference)
import jax
import jax.numpy as jnp
from jax.experimental import pallas as pl


def kernel(embeddings, predictions, edge_index, W1, b1, W2, b2, Wg, att_src, att_dst, bg):
    raise NotImplementedError("write your pallas kernel here")


# speedup vs baseline: 17.2203x; 17.2203x over previous
import math

import jax
import jax.numpy as jnp
from jax import lax
from jax.experimental import pallas as pl
from jax.experimental.pallas import tpu as pltpu
from jax.experimental.pallas import tpu_sc as plsc

N = 50000
E = 800000
HID = 128
C = 10
HEADS = 4

NSUB = 16
NCORE = 2
NP = 50048
RPS = NP // NSUB
K = 80
CHUNKS = E // K
CPS = CHUNKS // NSUB
AW = 28

BLK = 2000


def _prologue_body(pred_ref, vsrc_ref, vdst_ref, src_ref, adst_ref):
    logits = pred_ref[...]
    m = jnp.max(logits, axis=-1, keepdims=True)
    ex = jnp.exp(logits - m)
    probs = ex / jnp.sum(ex, axis=-1, keepdims=True)
    lab = jnp.argmax(logits, axis=-1).astype(jnp.float32)[:, None]
    asrc = jnp.dot(probs, vsrc_ref[...].T)
    adst = jnp.dot(probs, vdst_ref[...].T)
    src_ref[...] = jnp.concatenate(
        [probs, asrc, lab, jnp.zeros_like(lab)], axis=-1)
    adst_ref[...] = adst


def _sc_edge_body(*refs):
    cols = refs[:19]
    esrc, edst, ssrc, bnd_d, bnd_s = refs[19:24]
    acc0_o, acc1_o = refs[24], refs[25]
    sidx, didx, bbuf, gbuf, accT, sem = refs[26:]
    cid = lax.axis_index("c")
    wid = lax.axis_index("s")
    f32 = jnp.float32
    iota16 = lax.iota(jnp.int32, 16)
    zero16 = jnp.zeros((16,), f32)

    def _gather16(v, idx):
        dnums = lax.GatherDimensionNumbers(
            offset_dims=(), collapsed_slice_dims=(0,), start_index_map=(0,))
        return lax.gather(v, idx[:, None], dnums, (1,),
                          mode=lax.GatherScatterMode.PROMISE_IN_BOUNDS)

    nbrs = [jnp.minimum(iota16 + k, 15) for k in (1, 2, 4, 8)]
    guards = [iota16 <= 15 - k for k in (1, 2, 4, 8)]
    prev_l = jnp.maximum(iota16 - 1, 0)

    def _za(i, _):
        accT[pl.ds(i * 16, 16)] = zero16
        return 0
    lax.fori_loop(0, RPS * AW // 16, _za, 0)

    lo = wid * RPS

    def _seg_masks(key):
        takes = [(_gather16(key, nbrs[t]) == key) & guards[t]
                 for t in range(4)]
        pk = _gather16(key, prev_l)
        runstart = (iota16 == 0) | (pk != key)
        return takes, runstart

    def _seg_sum(v, takes):
        for t in range(4):
            vn = _gather16(v, nbrs[t])
            v = v + jnp.where(takes[t], vn, 0.0)
        return v

    def _bounds(bnd):
        pltpu.sync_copy(bnd.at[pl.ds(0, 32)], bbuf)
        sel = iota16 == wid
        b_lo = jnp.sum(jnp.where(sel, bbuf[pl.ds(0, 16)], 0))
        b_hi = jnp.sum(jnp.where(sel, bbuf[pl.ds(16, 16)], 0))
        return b_lo, b_hi

    e_lo, e_hi = _bounds(bnd_d)

    def _chunk(j, _):
        base = j * K
        pltpu.sync_copy(esrc.at[pl.ds(base, K)], sidx)
        pltpu.sync_copy(edst.at[pl.ds(base, K)], didx)
        descs = []
        for i in range(15):
            descs.append(pltpu.async_copy(cols[i].at[sidx], gbuf.at[i], sem))
        for i in range(15, 19):
            descs.append(pltpu.async_copy(cols[i].at[didx], gbuf.at[i], sem))
        for d in descs:
            d.wait()

        for g in range(K // 16):
            sl = pl.ds(g * 16, 16)
            dst = didx[sl]
            valid = (dst >= lo) & (dst < lo + RPS)
            dloc = dst - lo
            takes, runstart = _seg_masks(dst)
            wmask = valid & runstart
            labelf = gbuf[14, sl]
            ws = []
            for hh in range(2):
                a_s = jnp.where(cid == 0, gbuf[10 + hh, sl], gbuf[12 + hh, sl])
                a_d = jnp.where(cid == 0, gbuf[15 + hh, sl], gbuf[17 + hh, sl])
                s = a_s + a_d
                ws.append(jnp.exp(jnp.where(s >= 0.0, s, 0.2 * s)))

            base28 = dloc * AW

            def _acc(col, v):
                plsc.addupdate_scatter(accT, [base28 + col],
                                       _seg_sum(v, takes), mask=wmask)

            _acc(10, ws[0])
            _acc(21, ws[1])
            for c in range(C):
                p = gbuf[c, sl]
                _acc(c, p * ws[0])
                _acc(11 + c, p * ws[1])
            for c in range(5):
                tgt = (cid * 5 + c).astype(f32)
                _acc(22 + c, jnp.where(labelf == tgt, 1.0, 0.0))
        return 0

    lax.fori_loop(e_lo // K, (e_hi + K - 1) // K, _chunk, 0)

    @pl.when(cid == 1)
    def _():
        s_lo, s_hi = _bounds(bnd_s)

        def _chunk2(j, _):
            pltpu.sync_copy(ssrc.at[pl.ds(j * K, K)], sidx)
            for g in range(K // 16):
                srcv = sidx[pl.ds(g * 16, 16)]
                valid = (srcv >= lo) & (srcv < lo + RPS)
                takes, runstart = _seg_masks(srcv)
                cnt = _seg_sum(jnp.full((16,), 1.0, f32), takes)
                plsc.addupdate_scatter(accT, [(srcv - lo) * AW + 27], cnt,
                                       mask=valid & runstart)
            return 0

        lax.fori_loop(s_lo // K, (s_hi + K - 1) // K, _chunk2, 0)

    @pl.when(cid == 0)
    def _():
        pltpu.sync_copy(accT, acc0_o.at[pl.ds(lo * AW, RPS * AW)])

    @pl.when(cid == 1)
    def _():
        pltpu.sync_copy(accT, acc1_o.at[pl.ds(lo * AW, RPS * AW)])


def _epilogue_body(src_ref, adst_ref, emb_ref, a0_ref, a1_ref,
                   w1_ref, b1_ref, w2_ref, b2_ref, wg_ref, bg_ref,
                   lab_ref, comb_ref, fin_ref, maxp_ref, econf_ref,
                   gconf_ref, unc_ref):
    st = src_ref[...]
    probs = st[:, 0:10]
    asrc = st[:, 10:14]
    labelf = st[:, 14:15]
    adst = adst_ref[...]
    a0 = a0_ref[...]
    a1 = a1_ref[...]

    s = asrc + adst
    wself = jnp.exp(jnp.where(s >= 0.0, s, 0.2 * s))
    den = jnp.concatenate(
        [a0[:, 10:11], a0[:, 21:22], a1[:, 10:11], a1[:, 21:22]],
        axis=-1) + wself + 1e-16

    wg = wg_ref[...]
    gvp = jnp.zeros_like(probs)
    for h in range(HEADS):
        src_acc = a0 if h < 2 else a1
        lo = 0 if h % 2 == 0 else 11
        pw = src_acc[:, lo:lo + 10]
        ph = pw + wself[:, h:h + 1] * probs
        aggh = jnp.dot(ph, wg[h * 10:(h + 1) * 10, :].T)
        gvp = gvp + aggh / den[:, h:h + 1]
    gvp = 0.25 * gvp + bg_ref[...]

    pn = jnp.maximum(jnp.sqrt(jnp.sum(probs * probs, -1, keepdims=True)), 1e-8)
    gn = jnp.maximum(jnp.sqrt(jnp.sum(gvp * gvp, -1, keepdims=True)), 1e-8)
    cons = jnp.sum(probs * gvp, -1, keepdims=True) / (pn * gn)
    gconf = (cons + 1.0) * 0.5

    emb = emb_ref[...]
    h1 = jnp.maximum(jnp.dot(emb, w1_ref[...].T) + b1_ref[...], 0.0)
    z = jnp.sum(h1 * w2_ref[...], -1, keepdims=True) + b2_ref[0, 0]
    econf = 1.0 / (1.0 + jnp.exp(-z))

    maxp = jnp.max(probs, -1, keepdims=True)
    comb = 0.4 * maxp + 0.2 * econf + 0.2 + 0.2 * gconf

    labi = labelf.astype(jnp.int32)
    oh = jnp.where(
        lax.broadcasted_iota(jnp.int32, probs.shape, 1) == labi, 1.0, 0.0)
    nb = jnp.concatenate([a0[:, 22:27], a1[:, 22:27]], axis=-1) + oh
    degt = a1[:, 27:28] + 1.0
    own = jnp.sum(oh * nb, -1, keepdims=True) / (degt + 1e-8)
    fin = jnp.where((comb > 0.85) & (own >= 0.6), 1.0, 0.0)

    ent = -jnp.sum(probs * jnp.log(probs + 1e-8), -1, keepdims=True)

    lab_ref[...] = labi
    comb_ref[...] = comb
    fin_ref[...] = fin
    maxp_ref[...] = maxp
    econf_ref[...] = econf
    gconf_ref[...] = gconf
    unc_ref[...] = ent * (1.0 / math.log(C))


def _make_sc_call():
    f32 = jnp.float32
    mesh = plsc.VectorSubcoreMesh(
        core_axis_name="c", subcore_axis_name="s", num_cores=NCORE,
        num_subcores=NSUB)
    return pl.kernel(
        _sc_edge_body,
        out_type=[jax.ShapeDtypeStruct((NP * AW,), f32),
                  jax.ShapeDtypeStruct((NP * AW,), f32)],
        mesh=mesh,
        scratch_types=[
            pltpu.VMEM((K,), jnp.int32),
            pltpu.VMEM((K,), jnp.int32),
            pltpu.VMEM((32,), jnp.int32),
            pltpu.VMEM((19, K), f32),
            pltpu.VMEM((RPS * AW,), f32),
            pltpu.SemaphoreType.DMA,
        ],
        compiler_params=pltpu.CompilerParams(needs_layout_passes=False),
    )


def kernel(embeddings, predictions, edge_index, W1, b1, W2, b2, Wg,
           att_src, att_dst, bg):
    f32 = jnp.float32
    wg3 = Wg.reshape(HEADS, C, C)
    vsrc = jnp.einsum("hc,hck->hk", att_src, wg3)
    vdst = jnp.einsum("hc,hck->hk", att_dst, wg3)

    nblk = N // BLK
    full = lambda shape: pl.BlockSpec(shape, lambda i: (0,) * len(shape))
    rowb = lambda w: pl.BlockSpec((BLK, w), lambda i: (i, 0))

    srctab, adstt = pl.pallas_call(
        _prologue_body,
        grid=(nblk,),
        in_specs=[rowb(C), full((HEADS, C)), full((HEADS, C))],
        out_specs=[rowb(16), rowb(HEADS)],
        out_shape=[jax.ShapeDtypeStruct((N, 16), f32),
                   jax.ShapeDtypeStruct((N, HEADS), f32)],
    )(predictions, vsrc, vdst)

    order = jnp.argsort(edge_index[1])
    esrc_s = edge_index[0][order]
    edst_s = edge_index[1][order]
    ssrc = jnp.sort(edge_index[0])
    grid_pts = jnp.arange(0, NP + 1, RPS, dtype=jnp.int32)
    bd = jnp.searchsorted(edst_s, grid_pts).astype(jnp.int32)
    bs = jnp.searchsorted(ssrc, grid_pts).astype(jnp.int32)
    bnd_d = jnp.concatenate([bd[:16], bd[1:17]])
    bnd_s = jnp.concatenate([bs[:16], bs[1:17]])

    sc_edges = _make_sc_call()
    col_args = ([srctab[:, i] for i in range(15)]
                + [adstt[:, h] for h in range(HEADS)])
    acc0f, acc1f = sc_edges(*col_args, esrc_s, edst_s, ssrc, bnd_d, bnd_s)
    acc0 = acc0f.reshape(NP, AW)
    acc1 = acc1f.reshape(NP, AW)

    outs = pl.pallas_call(
        _epilogue_body,
        grid=(nblk,),
        in_specs=[rowb(16), rowb(HEADS), rowb(HID), rowb(AW), rowb(AW),
                  full((HID // 2, HID)), full((1, HID // 2)),
                  full((1, HID // 2)), full((1, 1)), full((HEADS * C, C)),
                  full((1, C))],
        out_specs=[rowb(1)] * 7,
        out_shape=[jax.ShapeDtypeStruct((N, 1), jnp.int32)]
        + [jax.ShapeDtypeStruct((N, 1), f32)] * 6,
    )(srctab, adstt, embeddings, acc0[:N], acc1[:N],
      W1, b1.reshape(1, -1), W2, b2.reshape(1, 1), Wg, bg.reshape(1, -1))
    lab, comb, fin, maxp, econf, gconf, unc = outs

    temporal = jnp.ones((N,), f32)
    return (lab[:, 0], comb[:, 0], fin[:, 0].astype(jnp.bool_), maxp[:, 0],
            econf[:, 0], temporal, gconf[:, 0], unc[:, 0])
